# baseline (speedup 1.0000x reference)
import functools

import jax
import jax.numpy as jnp
from jax.experimental import pallas as pl


def _router_block(x_ref, w_ref, o_ref):
    logits = jax.lax.dot_general(
        x_ref[...],
        w_ref[...],
        dimension_numbers=(((1,), (1,)), ((), ())),
        preferred_element_type=jnp.float32,
    )
    m = jnp.max(logits, axis=-1, keepdims=True)
    e = jnp.exp(logits - m)
    o_ref[...] = e / jnp.sum(e, axis=-1, keepdims=True)


@functools.partial(jax.jit, static_argnames=("block",))
def _router(x2d, W, block: int):
    n_tokens, d = x2d.shape
    n_experts = W.shape[0]
    grid = (n_tokens // block,)
    return pl.pallas_call(
        _router_block,
        grid=grid,
        in_specs=[
            pl.BlockSpec((block, d), lambda i: (i, 0)),
            pl.BlockSpec((n_experts, d), lambda i: (0, 0)),
        ],
        out_specs=pl.BlockSpec((block, n_experts), lambda i: (i, 0)),
        out_shape=jax.ShapeDtypeStruct((n_tokens, n_experts), jnp.float32),
    )(x2d, W)


def kernel(x, W):
    b, t, d = x.shape
    out = _router(x.reshape(b * t, d), W, block=512)
    return out.reshape(b, t, W.shape[0])


# speedup vs baseline: 1.1849x; 1.1849x over previous
import functools

import jax
import jax.numpy as jnp
from jax.experimental import pallas as pl
from jax.experimental.pallas import tpu as pltpu


def _router_block(x_ref, w_ref, o_ref):
    logits = jax.lax.dot_general(
        x_ref[...],
        w_ref[...],
        dimension_numbers=(((1,), (1,)), ((), ())),
        preferred_element_type=jnp.float32,
    )
    m = jnp.max(logits, axis=-1, keepdims=True)
    e = jnp.exp(logits - m)
    o_ref[...] = e / jnp.sum(e, axis=-1, keepdims=True)


@functools.partial(jax.jit, static_argnames=("block",))
def _router(x2d, W, block: int):
    n_tokens, d = x2d.shape
    n_experts = W.shape[0]
    grid = (n_tokens // block,)
    return pl.pallas_call(
        _router_block,
        grid=grid,
        in_specs=[
            pl.BlockSpec((block, d), lambda i: (i, 0)),
            pl.BlockSpec((n_experts, d), lambda i: (0, 0)),
        ],
        out_specs=pl.BlockSpec((block, n_experts), lambda i: (i, 0)),
        out_shape=jax.ShapeDtypeStruct((n_tokens, n_experts), jnp.float32),
        compiler_params=pltpu.CompilerParams(
            dimension_semantics=("parallel",),
            vmem_limit_bytes=100 * 1024 * 1024,
        ),
    )(x2d, W)


def kernel(x, W):
    b, t, d = x.shape
    out = _router(x.reshape(b * t, d), W, block=1024)
    return out.reshape(b, t, W.shape[0])
